# baseline (speedup 1.0000x reference)
import functools

import jax
import jax.numpy as jnp
from jax import lax
from jax.experimental import pallas as pl
from jax.experimental.pallas import tpu as pltpu, tpu_sc as plsc

D_MODEL = 768
D_VOCAB = 100000
V_PAD = 100352


_VB = 512


def _transpose_body(w_ref, t_ref):
    t_ref[...] = w_ref[...].T


def _transpose_table(W_E):
    grid = (V_PAD // _VB,)
    return pl.pallas_call(
        _transpose_body,
        grid=grid,
        in_specs=[pl.BlockSpec((D_MODEL, _VB), lambda j: (0, j))],
        out_specs=pl.BlockSpec((_VB, D_MODEL), lambda j: (j, 0)),
        out_shape=jax.ShapeDtypeStruct((V_PAD, D_MODEL), jnp.float32),
    )(W_E)


_B = 32768
_CHUNK = 64


def _make_gather():
    info = plsc.get_sparse_core_info()
    nc, ns = info.num_cores, info.num_subcores
    nw = nc * ns
    b_per_w = _B // nw
    n_steps = b_per_w // _CHUNK
    mesh = plsc.VectorSubcoreMesh(core_axis_name="c", subcore_axis_name="s")

    @functools.partial(
        pl.kernel,
        mesh=mesh,
        out_type=jax.ShapeDtypeStruct((_B, D_MODEL), jnp.float32),
        scratch_types=[
            pltpu.VMEM((_CHUNK,), jnp.int32),
            pltpu.VMEM((_CHUNK, D_MODEL), jnp.float32),
            pltpu.SemaphoreType.DMA,
        ],
    )
    def gather_k(table_hbm, idx_hbm, out_hbm, idx_v, rows_v, sem):
        wid = lax.axis_index("s") * nc + lax.axis_index("c")
        base = wid * b_per_w

        def body(i, carry):
            off = base + i * _CHUNK
            pltpu.sync_copy(idx_hbm.at[pl.ds(off, _CHUNK)], idx_v)
            pltpu.async_copy(table_hbm.at[idx_v], rows_v, sem).wait()
            pltpu.sync_copy(rows_v, out_hbm.at[pl.ds(off, _CHUNK)])
            return carry

        lax.fori_loop(0, n_steps, body, 0)

    return gather_k


def kernel(x, W_E):
    table = _transpose_table(W_E)
    xf = x.reshape(_B).astype(jnp.int32)
    out = _make_gather()(table, xf)
    return out.reshape(x.shape[0], x.shape[1], D_MODEL)


# speedup vs baseline: 6.0281x; 6.0281x over previous
import functools

import jax
import jax.numpy as jnp
from jax import lax
from jax.experimental import pallas as pl
from jax.experimental.pallas import tpu as pltpu, tpu_sc as plsc

D_MODEL = 768
D_VOCAB = 100000

_B = 32768
_CHUNK = 64


def _make_gather():
    info = plsc.get_sparse_core_info()
    nc, ns = info.num_cores, info.num_subcores
    nw = nc * ns
    b_per_w = _B // nw
    n_steps = b_per_w // _CHUNK
    mesh = plsc.VectorSubcoreMesh(core_axis_name="c", subcore_axis_name="s")

    @functools.partial(
        pl.kernel,
        mesh=mesh,
        out_type=jax.ShapeDtypeStruct((_B, D_MODEL), jnp.float32),
        scratch_types=[
            pltpu.VMEM((_CHUNK,), jnp.int32),
            pltpu.VMEM((_CHUNK, D_MODEL), jnp.float32),
            pltpu.SemaphoreType.DMA,
        ],
    )
    def gather_k(table_hbm, idx_hbm, out_hbm, idx_v, rows_v, sem):
        wid = lax.axis_index("s") * nc + lax.axis_index("c")
        base = wid * b_per_w

        def body(i, carry):
            off = base + i * _CHUNK
            pltpu.sync_copy(idx_hbm.at[pl.ds(off, _CHUNK)], idx_v)
            pltpu.async_copy(table_hbm.at[idx_v], rows_v, sem).wait()
            pltpu.sync_copy(rows_v, out_hbm.at[pl.ds(off, _CHUNK)])
            return carry

        lax.fori_loop(0, n_steps, body, 0)

    return gather_k


def kernel(x, W_E):
    xf = x.reshape(_B).astype(jnp.int32)
    out = _make_gather()(W_E.T, xf)
    return out.reshape(x.shape[0], x.shape[1], D_MODEL)


# speedup vs baseline: 6.7192x; 1.1146x over previous
import functools

import jax
import jax.numpy as jnp
from jax import lax
from jax.experimental import pallas as pl
from jax.experimental.pallas import tpu as pltpu, tpu_sc as plsc

D_MODEL = 768
D_VOCAB = 100000

_B = 32768
_CHUNK = 64


def _make_gather():
    info = plsc.get_sparse_core_info()
    nc, ns = info.num_cores, info.num_subcores
    nw = nc * ns
    b_per_w = _B // nw
    n_steps = b_per_w // _CHUNK
    mesh = plsc.VectorSubcoreMesh(core_axis_name="c", subcore_axis_name="s")

    @functools.partial(
        pl.kernel,
        mesh=mesh,
        out_type=jax.ShapeDtypeStruct((_B, D_MODEL), jnp.float32),
        scratch_types=[
            pltpu.VMEM((2, _CHUNK), jnp.int32),
            pltpu.VMEM((2, _CHUNK, D_MODEL), jnp.float32),
            pltpu.SemaphoreType.DMA,
            pltpu.SemaphoreType.DMA,
            pltpu.SemaphoreType.DMA,
            pltpu.SemaphoreType.DMA,
        ],
    )
    def gather_k(table_hbm, idx_hbm, out_hbm, idx_v, rows_v,
                 gsem0, gsem1, wsem0, wsem1):
        gsems = (gsem0, gsem1)
        wsems = (wsem0, wsem1)
        wid = lax.axis_index("s") * nc + lax.axis_index("c")
        base = wid * b_per_w

        def start_gather(i, b):
            pltpu.sync_copy(idx_hbm.at[pl.ds(base + i * _CHUNK, _CHUNK)],
                            idx_v.at[b])
            pltpu.async_copy(table_hbm.at[idx_v.at[b]], rows_v.at[b], gsems[b])

        start_gather(0, 0)

        def pair_body(p, carry):
            for b in range(2):
                i = p * 2 + b
                nb = 1 - b

                @pl.when(i > 0)
                def _():
                    pltpu.make_async_copy(
                        rows_v.at[nb],
                        out_hbm.at[pl.ds(base + (i - 1) * _CHUNK, _CHUNK)],
                        wsems[nb]).wait()

                pltpu.make_async_copy(table_hbm.at[idx_v.at[b]],
                                      rows_v.at[b], gsems[b]).wait()

                @pl.when(i + 1 < n_steps)
                def _():
                    start_gather(i + 1, nb)

                pltpu.async_copy(rows_v.at[b],
                                 out_hbm.at[pl.ds(base + i * _CHUNK, _CHUNK)],
                                 wsems[b])
            return carry

        lax.fori_loop(0, n_steps // 2, pair_body, 0)
        pltpu.make_async_copy(
            rows_v.at[1],
            out_hbm.at[pl.ds(base + (n_steps - 1) * _CHUNK, _CHUNK)],
            wsems[1]).wait()

    return gather_k


def kernel(x, W_E):
    xf = x.reshape(_B).astype(jnp.int32)
    out = _make_gather()(W_E.T, xf)
    return out.reshape(x.shape[0], x.shape[1], D_MODEL)


# speedup vs baseline: 7.0722x; 1.0525x over previous
import functools

import jax
import jax.numpy as jnp
from jax import lax
from jax.experimental import pallas as pl
from jax.experimental.pallas import tpu as pltpu, tpu_sc as plsc

D_MODEL = 768
D_VOCAB = 100000

_B = 32768
_CHUNK = 32
_NBUF = 4
_LOOK = 2


def _make_gather():
    info = plsc.get_sparse_core_info()
    nc, ns = info.num_cores, info.num_subcores
    nw = nc * ns
    b_per_w = _B // nw
    n_steps = b_per_w // _CHUNK
    mesh = plsc.VectorSubcoreMesh(core_axis_name="c", subcore_axis_name="s")

    @functools.partial(
        pl.kernel,
        mesh=mesh,
        out_type=jax.ShapeDtypeStruct((_B, D_MODEL), jnp.float32),
        scratch_types=[
            pltpu.VMEM((b_per_w,), jnp.int32),
            pltpu.VMEM((_NBUF, _CHUNK, D_MODEL), jnp.float32),
        ] + [pltpu.SemaphoreType.DMA] * (2 * _NBUF),
    )
    def gather_k(table_hbm, idx_hbm, out_hbm, idx_v, rows_v, *sems):
        gsems = sems[:_NBUF]
        wsems = sems[_NBUF:]
        wid = lax.axis_index("s") * nc + lax.axis_index("c")
        base = wid * b_per_w

        pltpu.sync_copy(idx_hbm.at[pl.ds(base, b_per_w)], idx_v)

        def start_gather(i, b):
            pltpu.async_copy(
                table_hbm.at[idx_v.at[pl.ds(i * _CHUNK, _CHUNK)]],
                rows_v.at[b], gsems[b])

        def wait_gather(i, b):
            pltpu.make_async_copy(
                table_hbm.at[idx_v.at[pl.ds(i * _CHUNK, _CHUNK)]],
                rows_v.at[b], gsems[b]).wait()

        def start_write(i, b):
            pltpu.async_copy(
                rows_v.at[b],
                out_hbm.at[pl.ds(base + i * _CHUNK, _CHUNK)], wsems[b])

        def wait_write(i, b):
            pltpu.make_async_copy(
                rows_v.at[b],
                out_hbm.at[pl.ds(base + i * _CHUNK, _CHUNK)], wsems[b]).wait()

        for u in range(_LOOK):
            start_gather(u, u)

        def ring_body(p, carry):
            for u in range(_NBUF):
                j = p * _NBUF + u
                g = j + _LOOK
                bg = (u + _LOOK) % _NBUF

                @pl.when(g < n_steps)
                def _():
                    @pl.when(g >= _NBUF)
                    def _():
                        wait_write(g - _NBUF, bg)
                    start_gather(g, bg)

                wait_gather(j, u)
                start_write(j, u)
            return carry

        lax.fori_loop(0, n_steps // _NBUF, ring_body, 0)
        for u in range(_NBUF):
            wait_write(n_steps - _NBUF + u, (n_steps - _NBUF + u) % _NBUF)

    return gather_k


def kernel(x, W_E):
    xf = x.reshape(_B).astype(jnp.int32)
    out = _make_gather()(W_E.T, xf)
    return out.reshape(x.shape[0], x.shape[1], D_MODEL)


# speedup vs baseline: 7.1443x; 1.0102x over previous
import functools

import jax
import jax.numpy as jnp
from jax import lax
from jax.experimental import pallas as pl
from jax.experimental.pallas import tpu as pltpu, tpu_sc as plsc

D_MODEL = 768
D_VOCAB = 100000

_B = 32768
_CHUNK = 16
_NBUF = 8
_LOOK = 4


def _make_gather():
    info = plsc.get_sparse_core_info()
    nc, ns = info.num_cores, info.num_subcores
    nw = nc * ns
    b_per_w = _B // nw
    n_steps = b_per_w // _CHUNK
    mesh = plsc.VectorSubcoreMesh(core_axis_name="c", subcore_axis_name="s")

    @functools.partial(
        pl.kernel,
        mesh=mesh,
        out_type=jax.ShapeDtypeStruct((_B, D_MODEL), jnp.float32),
        scratch_types=[
            pltpu.VMEM((b_per_w,), jnp.int32),
            pltpu.VMEM((_NBUF, _CHUNK, D_MODEL), jnp.float32),
        ] + [pltpu.SemaphoreType.DMA] * (2 * _NBUF),
    )
    def gather_k(table_hbm, idx_hbm, out_hbm, idx_v, rows_v, *sems):
        gsems = sems[:_NBUF]
        wsems = sems[_NBUF:]
        wid = lax.axis_index("s") * nc + lax.axis_index("c")
        base = wid * b_per_w

        pltpu.sync_copy(idx_hbm.at[pl.ds(base, b_per_w)], idx_v)

        def start_gather(i, b):
            pltpu.async_copy(
                table_hbm.at[idx_v.at[pl.ds(i * _CHUNK, _CHUNK)]],
                rows_v.at[b], gsems[b])

        def wait_gather(i, b):
            pltpu.make_async_copy(
                table_hbm.at[idx_v.at[pl.ds(i * _CHUNK, _CHUNK)]],
                rows_v.at[b], gsems[b]).wait()

        def start_write(i, b):
            pltpu.async_copy(
                rows_v.at[b],
                out_hbm.at[pl.ds(base + i * _CHUNK, _CHUNK)], wsems[b])

        def wait_write(i, b):
            pltpu.make_async_copy(
                rows_v.at[b],
                out_hbm.at[pl.ds(base + i * _CHUNK, _CHUNK)], wsems[b]).wait()

        for u in range(_LOOK):
            start_gather(u, u)

        def ring_body(p, carry):
            for u in range(_NBUF):
                j = p * _NBUF + u
                g = j + _LOOK
                bg = (u + _LOOK) % _NBUF

                @pl.when(g < n_steps)
                def _():
                    @pl.when(g >= _NBUF)
                    def _():
                        wait_write(g - _NBUF, bg)
                    start_gather(g, bg)

                wait_gather(j, u)
                start_write(j, u)
            return carry

        lax.fori_loop(0, n_steps // _NBUF, ring_body, 0)
        for u in range(_NBUF):
            wait_write(n_steps - _NBUF + u, (n_steps - _NBUF + u) % _NBUF)

    return gather_k


def kernel(x, W_E):
    xf = x.reshape(_B).astype(jnp.int32)
    out = _make_gather()(W_E.T, xf)
    return out.reshape(x.shape[0], x.shape[1], D_MODEL)
